# baseline (speedup 1.0000x reference)
import functools
import jax
import jax.numpy as jnp
from jax import lax
from jax.experimental import pallas as pl
from jax.experimental.pallas import tpu as pltpu
from jax.experimental.pallas import tpu_sc as plsc

E = 8
K = 2
H = 1024
F = 2048
T = 2048
BLK = 256
NT = 23
PADDED = NT * BLK
NW = 32
NB = T // NW
SUB = 16


def _router_body(x_ref, wg_ref, bg_ref, dw_ref, di_ref, te_ref):
    x = x_ref[...]
    logits = jnp.dot(x, wg_ref[...], preferred_element_type=jnp.float32)
    logits = logits + bg_ref[...]
    lane = lax.broadcasted_iota(jnp.int32, (T, 128), 1)
    valid = lane < E
    logits = jnp.where(valid, logits, -1e30)
    m = jnp.max(logits, axis=1, keepdims=True)
    ex = jnp.where(valid, jnp.exp(logits - m), 0.0)
    p = ex / jnp.sum(ex, axis=1, keepdims=True)

    w0 = jnp.max(p, axis=1, keepdims=True)
    e0 = jnp.min(jnp.where((p == w0) & valid, lane, 999), axis=1, keepdims=True)
    p2 = jnp.where(lane == e0, -1.0, p)
    w1 = jnp.max(p2, axis=1, keepdims=True)
    e1 = jnp.min(jnp.where((p2 == w1) & valid, lane, 999), axis=1, keepdims=True)

    oh0 = ((lane == e0) & valid).astype(jnp.float32)
    oh1 = ((lane == e1) & valid).astype(jnp.float32)

    r = lax.broadcasted_iota(jnp.int32, (128, 128), 0)
    c = lax.broadcasted_iota(jnp.int32, (128, 128), 1)
    lt = (c < r).astype(jnp.float32)

    def excl_cumsum(oh):
        parts = []
        carry = jnp.zeros((1, 128), jnp.float32)
        for ch in range(T // 128):
            blk = oh[ch * 128:(ch + 1) * 128, :]
            parts.append(jnp.dot(lt, blk, preferred_element_type=jnp.float32) + carry)
            carry = carry + jnp.sum(blk, axis=0, keepdims=True)
        return jnp.concatenate(parts, axis=0), carry

    r0, c0 = excl_cumsum(oh0)
    r1, c1 = excl_cumsum(oh1)

    counts = c0 + c1
    pc = jnp.floor((counts + (BLK - 1)) * (1.0 / BLK)) * BLK
    ut = (r < c).astype(jnp.float32)
    base = jnp.dot(pc, ut, preferred_element_type=jnp.float32)

    d0 = jnp.sum(oh0 * (base + r0), axis=1, keepdims=True)
    d1 = jnp.sum(oh1 * (base + c0 + r1), axis=1, keepdims=True)

    lane0 = lane == 0
    lane1 = lane == 1
    dw_ref[...] = jnp.where(lane0, w0, jnp.where(lane1, w1, 0.0))
    di_ref[...] = jnp.where(lane0, d0.astype(jnp.int32),
                            jnp.where(lane1, d1.astype(jnp.int32), 0))

    base_t = base * (1.0 / BLK)
    total_t = jnp.sum(pc, axis=1, keepdims=True) * (1.0 / BLK)
    lane_row = lax.broadcasted_iota(jnp.int32, (1, 128), 1)
    tl = lax.broadcasted_iota(jnp.int32, (8, 128), 1).astype(jnp.float32)
    acc = jnp.zeros((8, 128), jnp.float32)
    for e in range(E):
        b_e = jnp.sum(jnp.where(lane_row == e, base_t, 0.0), axis=1, keepdims=True)
        acc = acc + (b_e <= tl).astype(jnp.float32)
    texp = jnp.maximum(acc - 1.0, 0.0)

    def next_present(cur):
        nxt = jnp.full((8, 128), float(E))
        for e in range(E):
            p_e = jnp.sum(jnp.where(lane_row == e, counts, 0.0), axis=1, keepdims=True)
            nxt = jnp.minimum(nxt, jnp.where((p_e > 0) & (cur < e), float(e), float(E)))
        return nxt

    n1 = next_present(texp)
    n2 = next_present(n1)
    rowid = lax.broadcasted_iota(jnp.int32, (8, 128), 0)
    te_sent = jnp.where(tl < total_t, texp, float(E))
    vals = jnp.where(rowid == 0, te_sent, jnp.where(rowid == 1, n1, n2))
    te_ref[...] = vals.astype(jnp.int32)


def _run_router(x, wg_pad, bg_pad):
    return pl.pallas_call(
        _router_body,
        out_shape=[
            jax.ShapeDtypeStruct((T, 128), jnp.float32),
            jax.ShapeDtypeStruct((T, 128), jnp.int32),
            jax.ShapeDtypeStruct((8, 128), jnp.int32),
        ],
    )(x, wg_pad, bg_pad)


def _dispatch_body(x_hbm, d0_hbm, d1_hbm, xs_hbm, rows_v, i0_v, i1_v, semx, sem):
    wid = lax.axis_index("s") * 2 + lax.axis_index("c")
    base = wid * NB
    cx = pltpu.async_copy(x_hbm.at[pl.ds(base, NB)], rows_v, semx)
    pltpu.sync_copy(d0_hbm.at[pl.ds(base, NB)], i0_v)
    pltpu.sync_copy(d1_hbm.at[pl.ds(base, NB)], i1_v)
    cx.wait()
    c0 = pltpu.async_copy(rows_v, xs_hbm.at[i0_v], sem)
    c1 = pltpu.async_copy(rows_v, xs_hbm.at[i1_v], sem)
    c0.wait()
    c1.wait()


@functools.cache
def _make_dispatch():
    return pl.kernel(
        _dispatch_body,
        out_type=jax.ShapeDtypeStruct((PADDED, H), jnp.float32),
        compiler_params=pltpu.CompilerParams(
            skip_device_barrier=True, disable_bounds_checks=True),
        mesh=plsc.VectorSubcoreMesh(core_axis_name="c", subcore_axis_name="s"),
        scratch_types=[
            pltpu.VMEM((NB, H), jnp.float32),
            pltpu.VMEM((NB,), jnp.int32),
            pltpu.VMEM((NB,), jnp.int32),
            pltpu.SemaphoreType.DMA,
            pltpu.SemaphoreType.DMA,
        ],
    )


NBUF = 3


def _ffn_body(te_ref, n1_ref, n2_ref, xs_ref, b1_ref, b2_ref, w1_any, w2_any,
              out_ref, w1_scr, w2_scr, sem1, sem2, ord_s):
    t = pl.program_id(0)
    e = te_ref[t]
    valid = e < E

    @pl.when(t == 0)
    def _():
        ord_s[0] = 0
        pltpu.make_async_copy(w1_any.at[e], w1_scr.at[0], sem1.at[0]).start()
        pltpu.make_async_copy(w2_any.at[e], w2_scr.at[0], sem2.at[0]).start()
        en = n1_ref[0]

        @pl.when(en < E)
        def __():
            pltpu.make_async_copy(w1_any.at[en], w1_scr.at[1], sem1.at[1]).start()
            pltpu.make_async_copy(w2_any.at[en], w2_scr.at[1], sem2.at[1]).start()

    prev = te_ref[jnp.maximum(t - 1, 0)]
    boundary = jnp.logical_and(valid, jnp.logical_or(t == 0, prev != e))

    @pl.when(boundary)
    def _():
        o = ord_s[0]
        bi = lax.rem(o, NBUF)
        pltpu.make_async_copy(w1_any.at[e], w1_scr.at[bi], sem1.at[bi]).wait()
        pltpu.make_async_copy(w2_any.at[e], w2_scr.at[bi], sem2.at[bi]).wait()
        nn = n2_ref[t]

        @pl.when(nn < E)
        def __():
            bj = lax.rem(o + 2, NBUF)
            pltpu.make_async_copy(w1_any.at[nn], w1_scr.at[bj], sem1.at[bj]).start()
            pltpu.make_async_copy(w2_any.at[nn], w2_scr.at[bj], sem2.at[bj]).start()

        ord_s[0] = o + 1

    @pl.when(valid)
    def _():
        bi = lax.rem(ord_s[0] - 1, NBUF)
        h = jnp.dot(xs_ref[...], w1_scr[bi], preferred_element_type=jnp.float32)
        h = jnp.maximum(h + b1_ref[0], 0.0)
        y = jnp.dot(h, w2_scr[bi], preferred_element_type=jnp.float32)
        out_ref[...] = y + b2_ref[0]


def _run_ffn(te, n1, n2, xs, w1, b1, w2, b2):
    grid_spec = pltpu.PrefetchScalarGridSpec(
        num_scalar_prefetch=3,
        grid=(NT,),
        in_specs=[
            pl.BlockSpec((BLK, H), lambda t, te, n1, n2: (t, 0)),
            pl.BlockSpec((1, 1, F), lambda t, te, n1, n2: (jnp.minimum(te[t], E - 1), 0, 0)),
            pl.BlockSpec((1, 1, H), lambda t, te, n1, n2: (jnp.minimum(te[t], E - 1), 0, 0)),
            pl.BlockSpec(memory_space=pltpu.HBM),
            pl.BlockSpec(memory_space=pltpu.HBM),
        ],
        out_specs=pl.BlockSpec((BLK, H), lambda t, te, n1, n2: (t, 0)),
        scratch_shapes=[
            pltpu.VMEM((NBUF, H, F), jnp.float32),
            pltpu.VMEM((NBUF, F, H), jnp.float32),
            pltpu.SemaphoreType.DMA((NBUF,)),
            pltpu.SemaphoreType.DMA((NBUF,)),
            pltpu.SMEM((1,), jnp.int32),
        ],
    )
    return pl.pallas_call(
        _ffn_body,
        grid_spec=grid_spec,
        out_shape=jax.ShapeDtypeStruct((PADDED, H), jnp.float32),
        compiler_params=pltpu.CompilerParams(vmem_limit_bytes=110 * 1024 * 1024),
    )(te, n1, n2, xs, b1, b2, w1, w2)


def _combine_body(y_hbm, d0_hbm, d1_hbm, w0_hbm, w1_hbm, out_hbm,
                  r0a_v, r0b_v, r1a_v, r1b_v, i0a_v, i0b_v, i1a_v, i1b_v,
                  w0_v, w1_v, semga, semgb, semoa, semob):
    wid = lax.axis_index("s") * 2 + lax.axis_index("c")
    base = wid * NB
    nsub = NB // SUB
    r0 = [r0a_v, r0b_v]
    r1 = [r1a_v, r1b_v]
    i0 = [i0a_v, i0b_v]
    i1 = [i1a_v, i1b_v]
    semg = [semga, semgb]
    semo = [semoa, semob]
    pltpu.sync_copy(w0_hbm.at[pl.ds(base, NB)], w0_v)
    pltpu.sync_copy(w1_hbm.at[pl.ds(base, NB)], w1_v)

    def issue(s):
        pb = s % 2
        pltpu.sync_copy(d0_hbm.at[pl.ds(base + s * SUB, SUB)], i0[pb])
        pltpu.sync_copy(d1_hbm.at[pl.ds(base + s * SUB, SUB)], i1[pb])
        g0 = pltpu.async_copy(y_hbm.at[i0[pb]], r0[pb], semg[pb])
        g1 = pltpu.async_copy(y_hbm.at[i1[pb]], r1[pb], semg[pb])
        return g0, g1

    gh = [None] * nsub
    oh = [None] * nsub
    gh[0] = issue(0)
    gh[1] = issue(1)
    for s in range(nsub):
        pb = s % 2
        gh[s][0].wait()
        gh[s][1].wait()

        def row_fn(i, _):
            wa = plsc.load_gather(w0_v, [jnp.full((16,), s * SUB, jnp.int32) + i])
            wb = plsc.load_gather(w1_v, [jnp.full((16,), s * SUB, jnp.int32) + i])
            for cc in range(H // 16):
                a = r0[pb][i, pl.ds(cc * 16, 16)]
                bb = r1[pb][i, pl.ds(cc * 16, 16)]
                r0[pb][i, pl.ds(cc * 16, 16)] = a * wa + bb * wb
            return 0

        lax.fori_loop(0, SUB, row_fn, 0)
        oh[s] = pltpu.async_copy(r0[pb], out_hbm.at[pl.ds(base + s * SUB, SUB)], semo[pb])
        if s + 2 < nsub:
            oh[s].wait()
            gh[s + 2] = issue(s + 2)
    oh[nsub - 2].wait()
    oh[nsub - 1].wait()


@functools.cache
def _make_combine():
    return pl.kernel(
        _combine_body,
        out_type=jax.ShapeDtypeStruct((T, H), jnp.float32),
        compiler_params=pltpu.CompilerParams(
            needs_layout_passes=False, skip_device_barrier=True,
            disable_bounds_checks=True),
        mesh=plsc.VectorSubcoreMesh(core_axis_name="c", subcore_axis_name="s"),
        scratch_types=[
            pltpu.VMEM((SUB, H), jnp.float32),
            pltpu.VMEM((SUB, H), jnp.float32),
            pltpu.VMEM((SUB, H), jnp.float32),
            pltpu.VMEM((SUB, H), jnp.float32),
            pltpu.VMEM((SUB,), jnp.int32),
            pltpu.VMEM((SUB,), jnp.int32),
            pltpu.VMEM((SUB,), jnp.int32),
            pltpu.VMEM((SUB,), jnp.int32),
            pltpu.VMEM((NB,), jnp.float32),
            pltpu.VMEM((NB,), jnp.float32),
            pltpu.SemaphoreType.DMA,
            pltpu.SemaphoreType.DMA,
            pltpu.SemaphoreType.DMA,
            pltpu.SemaphoreType.DMA,
        ],
    )


@jax.jit
def kernel(input_tensor, Wg, bg, W1, b1, W2, b2):
    B, S, _ = input_tensor.shape
    x = input_tensor.reshape(T, H)
    wg_pad = jnp.zeros((H, 128), jnp.float32).at[:, :E].set(Wg)
    bg_pad = jnp.zeros((1, 128), jnp.float32).at[:, :E].set(bg)

    dw, di, te = _run_router(x, wg_pad, bg_pad)
    w0 = dw[:, 0]
    w1 = dw[:, 1]
    d0 = di[:, 0]
    d1 = di[:, 1]
    te_arr = te[0, :NT]
    n1_arr = te[1, :NT]
    n2_arr = te[2, :NT]

    xs = _make_dispatch()(x, d0, d1)
    ys = _run_ffn(te_arr, n1_arr, n2_arr, xs, W1, b1.reshape(E, 1, F),
                  W2, b2.reshape(E, 1, H))
    out = _make_combine()(ys, d0, d1, w0, w1)
    return out.reshape(B, S, H)


# speedup vs baseline: 1.0261x; 1.0261x over previous
import functools
import jax
import jax.numpy as jnp
from jax import lax
from jax.experimental import pallas as pl
from jax.experimental.pallas import tpu as pltpu
from jax.experimental.pallas import tpu_sc as plsc

E = 8
K = 2
H = 1024
F = 2048
T = 2048
BLK = 256
NT = 23
PADDED = NT * BLK
NW = 32
NB = T // NW
SUB = 16


def _router_body(x_ref, wg_ref, bg_ref, dw_ref, di_ref, te_ref):
    x = x_ref[...]
    wg = jnp.concatenate([wg_ref[...], jnp.zeros((H, 128 - E), jnp.float32)], axis=1)
    bg = jnp.concatenate([bg_ref[...], jnp.zeros((1, 128 - E), jnp.float32)], axis=1)
    logits = jnp.dot(x, wg, preferred_element_type=jnp.float32)
    logits = logits + bg
    lane = lax.broadcasted_iota(jnp.int32, (T, 128), 1)
    valid = lane < E
    logits = jnp.where(valid, logits, -1e30)
    m = jnp.max(logits, axis=1, keepdims=True)
    ex = jnp.where(valid, jnp.exp(logits - m), 0.0)
    p = ex / jnp.sum(ex, axis=1, keepdims=True)

    w0 = jnp.max(p, axis=1, keepdims=True)
    e0 = jnp.min(jnp.where((p == w0) & valid, lane, 999), axis=1, keepdims=True)
    p2 = jnp.where(lane == e0, -1.0, p)
    w1 = jnp.max(p2, axis=1, keepdims=True)
    e1 = jnp.min(jnp.where((p2 == w1) & valid, lane, 999), axis=1, keepdims=True)

    oh0 = ((lane == e0) & valid).astype(jnp.float32)
    oh1 = ((lane == e1) & valid).astype(jnp.float32)

    r = lax.broadcasted_iota(jnp.int32, (128, 128), 0)
    c = lax.broadcasted_iota(jnp.int32, (128, 128), 1)
    lt = (c < r).astype(jnp.float32)

    def excl_cumsum(oh):
        parts = []
        carry = jnp.zeros((1, 128), jnp.float32)
        for ch in range(T // 128):
            blk = oh[ch * 128:(ch + 1) * 128, :]
            parts.append(jnp.dot(lt, blk, preferred_element_type=jnp.float32) + carry)
            carry = carry + jnp.sum(blk, axis=0, keepdims=True)
        return jnp.concatenate(parts, axis=0), carry

    r0, c0 = excl_cumsum(oh0)
    r1, c1 = excl_cumsum(oh1)

    counts = c0 + c1
    pc = jnp.floor((counts + (BLK - 1)) * (1.0 / BLK)) * BLK
    ut = (r < c).astype(jnp.float32)
    base = jnp.dot(pc, ut, preferred_element_type=jnp.float32)

    d0 = jnp.sum(oh0 * (base + r0), axis=1, keepdims=True)
    d1 = jnp.sum(oh1 * (base + c0 + r1), axis=1, keepdims=True)

    lane0 = lane == 0
    lane1 = lane == 1
    dw_ref[...] = jnp.where(lane0, w0, jnp.where(lane1, w1, 0.0))
    di_ref[...] = jnp.where(lane0, d0.astype(jnp.int32),
                            jnp.where(lane1, d1.astype(jnp.int32), 0))

    base_t = base * (1.0 / BLK)
    total_t = jnp.sum(pc, axis=1, keepdims=True) * (1.0 / BLK)
    lane_row = lax.broadcasted_iota(jnp.int32, (1, 128), 1)
    tl = lax.broadcasted_iota(jnp.int32, (8, 128), 1).astype(jnp.float32)
    acc = jnp.zeros((8, 128), jnp.float32)
    for e in range(E):
        b_e = jnp.sum(jnp.where(lane_row == e, base_t, 0.0), axis=1, keepdims=True)
        acc = acc + (b_e <= tl).astype(jnp.float32)
    texp = jnp.maximum(acc - 1.0, 0.0)

    def next_present(cur):
        nxt = jnp.full((8, 128), float(E))
        for e in range(E):
            p_e = jnp.sum(jnp.where(lane_row == e, counts, 0.0), axis=1, keepdims=True)
            nxt = jnp.minimum(nxt, jnp.where((p_e > 0) & (cur < e), float(e), float(E)))
        return nxt

    n1 = next_present(texp)
    n2 = next_present(n1)
    rowid = lax.broadcasted_iota(jnp.int32, (8, 128), 0)
    te_sent = jnp.where(tl < total_t, texp, float(E))
    vals = jnp.where(rowid == 0, te_sent, jnp.where(rowid == 1, n1, n2))
    te_ref[...] = vals.astype(jnp.int32)


def _run_router(x, wg_pad, bg_pad):
    return pl.pallas_call(
        _router_body,
        out_shape=[
            jax.ShapeDtypeStruct((T, 128), jnp.float32),
            jax.ShapeDtypeStruct((T, 128), jnp.int32),
            jax.ShapeDtypeStruct((8, 128), jnp.int32),
        ],
    )(x, wg_pad, bg_pad)


def _dispatch_body(x_hbm, d0_hbm, d1_hbm, xs_hbm, rows_v, i0_v, i1_v, semx, sem):
    wid = lax.axis_index("s") * 2 + lax.axis_index("c")
    base = wid * NB
    cx = pltpu.async_copy(x_hbm.at[pl.ds(base, NB)], rows_v, semx)
    pltpu.sync_copy(d0_hbm.at[pl.ds(base, NB)], i0_v)
    pltpu.sync_copy(d1_hbm.at[pl.ds(base, NB)], i1_v)
    cx.wait()
    c0 = pltpu.async_copy(rows_v, xs_hbm.at[i0_v], sem)
    c1 = pltpu.async_copy(rows_v, xs_hbm.at[i1_v], sem)
    c0.wait()
    c1.wait()


@functools.cache
def _make_dispatch():
    return pl.kernel(
        _dispatch_body,
        out_type=jax.ShapeDtypeStruct((PADDED, H), jnp.float32),
        mesh=plsc.VectorSubcoreMesh(core_axis_name="c", subcore_axis_name="s"),
        scratch_types=[
            pltpu.VMEM((NB, H), jnp.float32),
            pltpu.VMEM((NB,), jnp.int32),
            pltpu.VMEM((NB,), jnp.int32),
            pltpu.SemaphoreType.DMA,
            pltpu.SemaphoreType.DMA,
        ],
    )


NBUF = 3


def _ffn_body(te_ref, n1_ref, n2_ref, xs_ref, b1_ref, b2_ref, w1_any, w2_any,
              out_ref, w1_scr, w2_scr, sem1, sem2, ord_s):
    t = pl.program_id(0)
    e = te_ref[t]
    valid = e < E

    @pl.when(t == 0)
    def _():
        ord_s[0] = 0
        pltpu.make_async_copy(w1_any.at[e], w1_scr.at[0], sem1.at[0]).start()
        pltpu.make_async_copy(w2_any.at[e], w2_scr.at[0], sem2.at[0]).start()
        en = n1_ref[0]

        @pl.when(en < E)
        def __():
            pltpu.make_async_copy(w1_any.at[en], w1_scr.at[1], sem1.at[1]).start()
            pltpu.make_async_copy(w2_any.at[en], w2_scr.at[1], sem2.at[1]).start()

    prev = te_ref[jnp.maximum(t - 1, 0)]
    boundary = jnp.logical_and(valid, jnp.logical_or(t == 0, prev != e))

    @pl.when(boundary)
    def _():
        o = ord_s[0]
        bi = lax.rem(o, NBUF)
        pltpu.make_async_copy(w1_any.at[e], w1_scr.at[bi], sem1.at[bi]).wait()
        pltpu.make_async_copy(w2_any.at[e], w2_scr.at[bi], sem2.at[bi]).wait()
        nn = n2_ref[t]

        @pl.when(nn < E)
        def __():
            bj = lax.rem(o + 2, NBUF)
            pltpu.make_async_copy(w1_any.at[nn], w1_scr.at[bj], sem1.at[bj]).start()
            pltpu.make_async_copy(w2_any.at[nn], w2_scr.at[bj], sem2.at[bj]).start()

        ord_s[0] = o + 1

    @pl.when(valid)
    def _():
        bi = lax.rem(ord_s[0] - 1, NBUF)
        h = jnp.dot(xs_ref[...], w1_scr[bi], preferred_element_type=jnp.float32)
        h = jnp.maximum(h + b1_ref[0], 0.0)
        y = jnp.dot(h, w2_scr[bi], preferred_element_type=jnp.float32)
        out_ref[...] = y + b2_ref[0]


def _run_ffn(te, n1, n2, xs, w1, b1, w2, b2):
    grid_spec = pltpu.PrefetchScalarGridSpec(
        num_scalar_prefetch=3,
        grid=(NT,),
        in_specs=[
            pl.BlockSpec((BLK, H), lambda t, te, n1, n2: (t, 0)),
            pl.BlockSpec((1, 1, F), lambda t, te, n1, n2: (jnp.minimum(te[t], E - 1), 0, 0)),
            pl.BlockSpec((1, 1, H), lambda t, te, n1, n2: (jnp.minimum(te[t], E - 1), 0, 0)),
            pl.BlockSpec(memory_space=pltpu.HBM),
            pl.BlockSpec(memory_space=pltpu.HBM),
        ],
        out_specs=pl.BlockSpec((BLK, H), lambda t, te, n1, n2: (t, 0)),
        scratch_shapes=[
            pltpu.VMEM((NBUF, H, F), jnp.float32),
            pltpu.VMEM((NBUF, F, H), jnp.float32),
            pltpu.SemaphoreType.DMA((NBUF,)),
            pltpu.SemaphoreType.DMA((NBUF,)),
            pltpu.SMEM((1,), jnp.int32),
        ],
    )
    return pl.pallas_call(
        _ffn_body,
        grid_spec=grid_spec,
        out_shape=jax.ShapeDtypeStruct((PADDED, H), jnp.float32),
        compiler_params=pltpu.CompilerParams(vmem_limit_bytes=110 * 1024 * 1024),
    )(te, n1, n2, xs, b1, b2, w1, w2)


def _combine_body(y_hbm, d0_hbm, d1_hbm, w0_hbm, w1_hbm, out_hbm,
                  r0a_v, r0b_v, r1a_v, r1b_v, i0a_v, i0b_v, i1a_v, i1b_v,
                  w0_v, w1_v, semga, semgb, semoa, semob):
    wid = lax.axis_index("s") * 2 + lax.axis_index("c")
    base = wid * NB
    nsub = NB // SUB
    r0 = [r0a_v, r0b_v]
    r1 = [r1a_v, r1b_v]
    i0 = [i0a_v, i0b_v]
    i1 = [i1a_v, i1b_v]
    semg = [semga, semgb]
    semo = [semoa, semob]
    pltpu.sync_copy(w0_hbm.at[pl.ds(base, NB)], w0_v)
    pltpu.sync_copy(w1_hbm.at[pl.ds(base, NB)], w1_v)

    def issue(s):
        pb = s % 2
        pltpu.sync_copy(d0_hbm.at[pl.ds(base + s * SUB, SUB)], i0[pb])
        pltpu.sync_copy(d1_hbm.at[pl.ds(base + s * SUB, SUB)], i1[pb])
        g0 = pltpu.async_copy(y_hbm.at[i0[pb]], r0[pb], semg[pb])
        g1 = pltpu.async_copy(y_hbm.at[i1[pb]], r1[pb], semg[pb])
        return g0, g1

    gh = [None] * nsub
    oh = [None] * nsub
    gh[0] = issue(0)
    gh[1] = issue(1)
    for s in range(nsub):
        pb = s % 2
        gh[s][0].wait()
        gh[s][1].wait()

        def row_fn(i, _):
            wa = plsc.load_gather(w0_v, [jnp.full((16,), s * SUB, jnp.int32) + i])
            wb = plsc.load_gather(w1_v, [jnp.full((16,), s * SUB, jnp.int32) + i])
            for cc in range(H // 16):
                a = r0[pb][i, pl.ds(cc * 16, 16)]
                bb = r1[pb][i, pl.ds(cc * 16, 16)]
                r0[pb][i, pl.ds(cc * 16, 16)] = a * wa + bb * wb
            return 0

        lax.fori_loop(0, SUB, row_fn, 0)
        oh[s] = pltpu.async_copy(r0[pb], out_hbm.at[pl.ds(base + s * SUB, SUB)], semo[pb])
        if s + 2 < nsub:
            oh[s].wait()
            gh[s + 2] = issue(s + 2)
    oh[nsub - 2].wait()
    oh[nsub - 1].wait()


@functools.cache
def _make_combine():
    return pl.kernel(
        _combine_body,
        out_type=jax.ShapeDtypeStruct((T, H), jnp.float32),
        compiler_params=pltpu.CompilerParams(needs_layout_passes=False),
        mesh=plsc.VectorSubcoreMesh(core_axis_name="c", subcore_axis_name="s"),
        scratch_types=[
            pltpu.VMEM((SUB, H), jnp.float32),
            pltpu.VMEM((SUB, H), jnp.float32),
            pltpu.VMEM((SUB, H), jnp.float32),
            pltpu.VMEM((SUB, H), jnp.float32),
            pltpu.VMEM((SUB,), jnp.int32),
            pltpu.VMEM((SUB,), jnp.int32),
            pltpu.VMEM((SUB,), jnp.int32),
            pltpu.VMEM((SUB,), jnp.int32),
            pltpu.VMEM((NB,), jnp.float32),
            pltpu.VMEM((NB,), jnp.float32),
            pltpu.SemaphoreType.DMA,
            pltpu.SemaphoreType.DMA,
            pltpu.SemaphoreType.DMA,
            pltpu.SemaphoreType.DMA,
        ],
    )


@jax.jit
def kernel(input_tensor, Wg, bg, W1, b1, W2, b2):
    B, S, _ = input_tensor.shape
    x = input_tensor.reshape(T, H)

    dw, di, te = _run_router(x, Wg, bg.reshape(1, E))
    w0 = dw[:, 0]
    w1 = dw[:, 1]
    d0 = di[:, 0]
    d1 = di[:, 1]
    te_arr = te[0, :NT]
    n1_arr = te[1, :NT]
    n2_arr = te[2, :NT]

    xs = _make_dispatch()(x, d0, d1)
    ys = _run_ffn(te_arr, n1_arr, n2_arr, xs, W1, b1.reshape(E, 1, F),
                  W2, b2.reshape(E, 1, H))
    out = _make_combine()(ys, d0, d1, w0, w1)
    return out.reshape(B, S, H)


# speedup vs baseline: 1.0385x; 1.0121x over previous
import functools
import jax
import jax.numpy as jnp
from jax import lax
from jax.experimental import pallas as pl
from jax.experimental.pallas import tpu as pltpu
from jax.experimental.pallas import tpu_sc as plsc

E = 8
K = 2
H = 1024
F = 2048
T = 2048
BLK = 256
NT = 23
PADDED = NT * BLK
NW = 32
NB = T // NW
SUB = 16


def _router_body(x_ref, wg_ref, bg_ref, dw_ref, di_ref, te_ref):
    x = x_ref[...]
    wg = jnp.concatenate([wg_ref[...], jnp.zeros((H, 128 - E), jnp.float32)], axis=1)
    bg = jnp.concatenate([bg_ref[...], jnp.zeros((1, 128 - E), jnp.float32)], axis=1)
    logits = jnp.dot(x, wg, preferred_element_type=jnp.float32)
    logits = logits + bg
    lane = lax.broadcasted_iota(jnp.int32, (T, 128), 1)
    valid = lane < E
    logits = jnp.where(valid, logits, -1e30)
    m = jnp.max(logits, axis=1, keepdims=True)
    ex = jnp.where(valid, jnp.exp(logits - m), 0.0)
    p = ex / jnp.sum(ex, axis=1, keepdims=True)

    w0 = jnp.max(p, axis=1, keepdims=True)
    e0 = jnp.min(jnp.where((p == w0) & valid, lane, 999), axis=1, keepdims=True)
    p2 = jnp.where(lane == e0, -1.0, p)
    w1 = jnp.max(p2, axis=1, keepdims=True)
    e1 = jnp.min(jnp.where((p2 == w1) & valid, lane, 999), axis=1, keepdims=True)

    oh0 = ((lane == e0) & valid).astype(jnp.float32)
    oh1 = ((lane == e1) & valid).astype(jnp.float32)

    r = lax.broadcasted_iota(jnp.int32, (128, 128), 0)
    c = lax.broadcasted_iota(jnp.int32, (128, 128), 1)
    lt = (c < r).astype(jnp.float32)

    def excl_cumsum(oh):
        parts = []
        carry = jnp.zeros((1, 128), jnp.float32)
        for ch in range(T // 128):
            blk = oh[ch * 128:(ch + 1) * 128, :]
            parts.append(jnp.dot(lt, blk, preferred_element_type=jnp.float32) + carry)
            carry = carry + jnp.sum(blk, axis=0, keepdims=True)
        return jnp.concatenate(parts, axis=0), carry

    r0, c0 = excl_cumsum(oh0)
    r1, c1 = excl_cumsum(oh1)

    counts = c0 + c1
    pc = jnp.floor((counts + (BLK - 1)) * (1.0 / BLK)) * BLK
    ut = (r < c).astype(jnp.float32)
    base = jnp.dot(pc, ut, preferred_element_type=jnp.float32)

    d0 = jnp.sum(oh0 * (base + r0), axis=1, keepdims=True)
    d1 = jnp.sum(oh1 * (base + c0 + r1), axis=1, keepdims=True)

    w0t = jnp.transpose(w0, (1, 0))
    w1t = jnp.transpose(w1, (1, 0))
    d0t = jnp.transpose(d0, (1, 0))
    d1t = jnp.transpose(d1, (1, 0))
    row8 = lax.broadcasted_iota(jnp.int32, (8, T), 0)
    dw_ref[...] = jnp.where(row8 == 0, w0t, jnp.where(row8 == 1, w1t, 0.0))
    di_ref[...] = jnp.where(row8 == 0, d0t, jnp.where(row8 == 1, d1t, 0.0)).astype(jnp.int32)

    base_t = base * (1.0 / BLK)
    total_t = jnp.sum(pc, axis=1, keepdims=True) * (1.0 / BLK)
    lane_row = lax.broadcasted_iota(jnp.int32, (1, 128), 1)
    tl = lax.broadcasted_iota(jnp.int32, (8, 128), 1).astype(jnp.float32)
    acc = jnp.zeros((8, 128), jnp.float32)
    for e in range(E):
        b_e = jnp.sum(jnp.where(lane_row == e, base_t, 0.0), axis=1, keepdims=True)
        acc = acc + (b_e <= tl).astype(jnp.float32)
    texp = jnp.maximum(acc - 1.0, 0.0)

    def next_present(cur):
        nxt = jnp.full((8, 128), float(E))
        for e in range(E):
            p_e = jnp.sum(jnp.where(lane_row == e, counts, 0.0), axis=1, keepdims=True)
            nxt = jnp.minimum(nxt, jnp.where((p_e > 0) & (cur < e), float(e), float(E)))
        return nxt

    n1 = next_present(texp)
    n2 = next_present(n1)
    rowid = lax.broadcasted_iota(jnp.int32, (8, 128), 0)
    te_sent = jnp.where(tl < total_t, texp, float(E))
    vals = jnp.where(rowid == 0, te_sent, jnp.where(rowid == 1, n1, n2))
    te_ref[...] = vals.astype(jnp.int32)


def _run_router(x, wg_pad, bg_pad):
    return pl.pallas_call(
        _router_body,
        out_shape=[
            jax.ShapeDtypeStruct((8, T), jnp.float32),
            jax.ShapeDtypeStruct((8, T), jnp.int32),
            jax.ShapeDtypeStruct((8, 128), jnp.int32),
        ],
    )(x, wg_pad, bg_pad)


def _dispatch_body(x_hbm, di_hbm, xs_hbm, rows_v, i0_v, i1_v, semx, sem):
    wid = lax.axis_index("s") * 2 + lax.axis_index("c")
    base = wid * NB
    cx = pltpu.async_copy(x_hbm.at[pl.ds(base, NB)], rows_v, semx)
    pltpu.sync_copy(di_hbm.at[0, pl.ds(base, NB)], i0_v)
    pltpu.sync_copy(di_hbm.at[1, pl.ds(base, NB)], i1_v)
    cx.wait()
    c0 = pltpu.async_copy(rows_v, xs_hbm.at[i0_v], sem)
    c1 = pltpu.async_copy(rows_v, xs_hbm.at[i1_v], sem)
    c0.wait()
    c1.wait()


@functools.cache
def _make_dispatch():
    return pl.kernel(
        _dispatch_body,
        out_type=jax.ShapeDtypeStruct((PADDED, H), jnp.float32),
        mesh=plsc.VectorSubcoreMesh(core_axis_name="c", subcore_axis_name="s"),
        scratch_types=[
            pltpu.VMEM((NB, H), jnp.float32),
            pltpu.VMEM((NB,), jnp.int32),
            pltpu.VMEM((NB,), jnp.int32),
            pltpu.SemaphoreType.DMA,
            pltpu.SemaphoreType.DMA,
        ],
    )


NBUF = 3


def _ffn_body(te_ref, xs_ref, b1_ref, b2_ref, w1_any, w2_any,
              out_ref, w1_scr, w2_scr, sem1, sem2, ord_s):
    t = pl.program_id(0)
    e = te_ref[t]
    valid = e < E

    @pl.when(t == 0)
    def _():
        ord_s[0] = 0
        pltpu.make_async_copy(w1_any.at[e], w1_scr.at[0], sem1.at[0]).start()
        pltpu.make_async_copy(w2_any.at[e], w2_scr.at[0], sem2.at[0]).start()
        en = te_ref[128]

        @pl.when(en < E)
        def __():
            pltpu.make_async_copy(w1_any.at[en], w1_scr.at[1], sem1.at[1]).start()
            pltpu.make_async_copy(w2_any.at[en], w2_scr.at[1], sem2.at[1]).start()

    prev = te_ref[jnp.maximum(t - 1, 0)]
    boundary = jnp.logical_and(valid, jnp.logical_or(t == 0, prev != e))

    @pl.when(boundary)
    def _():
        o = ord_s[0]
        bi = lax.rem(o, NBUF)
        pltpu.make_async_copy(w1_any.at[e], w1_scr.at[bi], sem1.at[bi]).wait()
        pltpu.make_async_copy(w2_any.at[e], w2_scr.at[bi], sem2.at[bi]).wait()
        nn = te_ref[256 + t]

        @pl.when(nn < E)
        def __():
            bj = lax.rem(o + 2, NBUF)
            pltpu.make_async_copy(w1_any.at[nn], w1_scr.at[bj], sem1.at[bj]).start()
            pltpu.make_async_copy(w2_any.at[nn], w2_scr.at[bj], sem2.at[bj]).start()

        ord_s[0] = o + 1

    @pl.when(valid)
    def _():
        bi = lax.rem(ord_s[0] - 1, NBUF)
        h = jnp.dot(xs_ref[...], w1_scr[bi], preferred_element_type=jnp.float32)
        h = jnp.maximum(h + b1_ref[0], 0.0)
        y = jnp.dot(h, w2_scr[bi], preferred_element_type=jnp.float32)
        out_ref[...] = y + b2_ref[0]


def _run_ffn(te, xs, w1, b1, w2, b2):
    grid_spec = pltpu.PrefetchScalarGridSpec(
        num_scalar_prefetch=1,
        grid=(NT,),
        in_specs=[
            pl.BlockSpec((BLK, H), lambda t, te: (t, 0)),
            pl.BlockSpec((1, 1, F), lambda t, te: (jnp.minimum(te[t], E - 1), 0, 0)),
            pl.BlockSpec((1, 1, H), lambda t, te: (jnp.minimum(te[t], E - 1), 0, 0)),
            pl.BlockSpec(memory_space=pltpu.HBM),
            pl.BlockSpec(memory_space=pltpu.HBM),
        ],
        out_specs=pl.BlockSpec((BLK, H), lambda t, te: (t, 0)),
        scratch_shapes=[
            pltpu.VMEM((NBUF, H, F), jnp.float32),
            pltpu.VMEM((NBUF, F, H), jnp.float32),
            pltpu.SemaphoreType.DMA((NBUF,)),
            pltpu.SemaphoreType.DMA((NBUF,)),
            pltpu.SMEM((1,), jnp.int32),
        ],
    )
    return pl.pallas_call(
        _ffn_body,
        grid_spec=grid_spec,
        out_shape=jax.ShapeDtypeStruct((PADDED, H), jnp.float32),
        compiler_params=pltpu.CompilerParams(vmem_limit_bytes=110 * 1024 * 1024),
    )(te, xs, b1, b2, w1, w2)


def _combine_body(y_hbm, di_hbm, dw_hbm, out_hbm,
                  r0a_v, r0b_v, r1a_v, r1b_v, i0a_v, i0b_v, i1a_v, i1b_v,
                  w0_v, w1_v, semga, semgb, semoa, semob):
    wid = lax.axis_index("s") * 2 + lax.axis_index("c")
    base = wid * NB
    nsub = NB // SUB
    r0 = [r0a_v, r0b_v]
    r1 = [r1a_v, r1b_v]
    i0 = [i0a_v, i0b_v]
    i1 = [i1a_v, i1b_v]
    semg = [semga, semgb]
    semo = [semoa, semob]
    pltpu.sync_copy(dw_hbm.at[0, pl.ds(base, NB)], w0_v)
    pltpu.sync_copy(dw_hbm.at[1, pl.ds(base, NB)], w1_v)

    def issue(s):
        pb = s % 2
        pltpu.sync_copy(di_hbm.at[0, pl.ds(base + s * SUB, SUB)], i0[pb])
        pltpu.sync_copy(di_hbm.at[1, pl.ds(base + s * SUB, SUB)], i1[pb])
        g0 = pltpu.async_copy(y_hbm.at[i0[pb]], r0[pb], semg[pb])
        g1 = pltpu.async_copy(y_hbm.at[i1[pb]], r1[pb], semg[pb])
        return g0, g1

    gh = [None] * nsub
    oh = [None] * nsub
    gh[0] = issue(0)
    gh[1] = issue(1)
    for s in range(nsub):
        pb = s % 2
        gh[s][0].wait()
        gh[s][1].wait()

        def row_fn(i, _):
            wa = plsc.load_gather(w0_v, [jnp.full((16,), s * SUB, jnp.int32) + i])
            wb = plsc.load_gather(w1_v, [jnp.full((16,), s * SUB, jnp.int32) + i])
            for cc in range(H // 16):
                a = r0[pb][i, pl.ds(cc * 16, 16)]
                bb = r1[pb][i, pl.ds(cc * 16, 16)]
                r0[pb][i, pl.ds(cc * 16, 16)] = a * wa + bb * wb
            return 0

        lax.fori_loop(0, SUB, row_fn, 0)
        oh[s] = pltpu.async_copy(r0[pb], out_hbm.at[pl.ds(base + s * SUB, SUB)], semo[pb])
        if s + 2 < nsub:
            oh[s].wait()
            gh[s + 2] = issue(s + 2)
    oh[nsub - 2].wait()
    oh[nsub - 1].wait()


@functools.cache
def _make_combine():
    return pl.kernel(
        _combine_body,
        out_type=jax.ShapeDtypeStruct((T, H), jnp.float32),
        compiler_params=pltpu.CompilerParams(needs_layout_passes=False),
        mesh=plsc.VectorSubcoreMesh(core_axis_name="c", subcore_axis_name="s"),
        scratch_types=[
            pltpu.VMEM((SUB, H), jnp.float32),
            pltpu.VMEM((SUB, H), jnp.float32),
            pltpu.VMEM((SUB, H), jnp.float32),
            pltpu.VMEM((SUB, H), jnp.float32),
            pltpu.VMEM((SUB,), jnp.int32),
            pltpu.VMEM((SUB,), jnp.int32),
            pltpu.VMEM((SUB,), jnp.int32),
            pltpu.VMEM((SUB,), jnp.int32),
            pltpu.VMEM((NB,), jnp.float32),
            pltpu.VMEM((NB,), jnp.float32),
            pltpu.SemaphoreType.DMA,
            pltpu.SemaphoreType.DMA,
            pltpu.SemaphoreType.DMA,
            pltpu.SemaphoreType.DMA,
        ],
    )


@jax.jit
def kernel(input_tensor, Wg, bg, W1, b1, W2, b2):
    B, S, _ = input_tensor.shape
    x = input_tensor.reshape(T, H)

    dw, di, te = _run_router(x, Wg, bg.reshape(1, E))

    xs = _make_dispatch()(x, di)
    ys = _run_ffn(te.reshape(8 * 128), xs, W1, b1.reshape(E, 1, F),
                  W2, b2.reshape(E, 1, H))
    out = _make_combine()(ys, di, dw)
    return out.reshape(B, S, H)
